# baseline (device time: 49312 ns/iter reference)
import jax
import jax.numpy as jnp
from jax import lax
from jax.experimental import pallas as pl
from jax.experimental.pallas import tpu as pltpu

N_Z = 4
P = 8


def kernel(x):
    m_per, n = x.shape
    half = m_per // 2
    rows = half // P

    def body(x_hbm, out_hbm, stg, x_vmem,
             dn_s, up_s, xl_s, dn_r, up_r, xl_r, in_sem, hbm_sem):
        my_x = lax.axis_index("x")
        my_y = lax.axis_index("y")
        my_z = lax.axis_index("z")
        partner = (1 - my_x, my_y, my_z)
        dn_nbr = (my_x, my_y, my_z - 1)
        up_nbr = (my_x, my_y, my_z + 1)
        ph = my_x * half
        qh = (1 - my_x) * half
        has_dn = my_z > 0
        has_up = my_z < N_Z - 1

        def clip(c):
            return jnp.clip(c, 0, N_Z - 1)

        in_cp = [pltpu.make_async_copy(
            x_hbm.at[pl.ds(off, half), :], x_vmem.at[pl.ds(off, half), :],
            in_sem.at[s]) for s, off in ((0, ph), (1, qh))]
        in_cp[0].start()
        in_cp[1].start()

        barrier = pltpu.get_barrier_semaphore()
        pl.semaphore_signal(barrier, inc=1, device_id=partner,
                            device_id_type=pl.DeviceIdType.MESH)

        @pl.when(has_dn)
        def _():
            pl.semaphore_signal(barrier, inc=1, device_id=dn_nbr,
                                device_id_type=pl.DeviceIdType.MESH)

        @pl.when(has_up)
        def _():
            pl.semaphore_signal(barrier, inc=1, device_id=up_nbr,
                                device_id_type=pl.DeviceIdType.MESH)

        in_cp[0].wait()
        stg[pl.ds(my_z * m_per + ph, half), :] = (
            x_vmem[pl.ds(ph, half), :].astype(jnp.bfloat16))

        interior = has_dn & has_up

        @pl.when(interior)
        def _():
            pl.semaphore_wait(barrier, 3)

        @pl.when(jnp.logical_not(interior))
        def _():
            pl.semaphore_wait(barrier, 2)

        def piece(chunk, off, i):
            r = clip(chunk) * m_per + off + i * rows
            return stg.at[pl.ds(r, rows), :]

        def desc(chunk_src, chunk_dst, off, i, ssem, rsem, target):
            return pltpu.make_async_remote_copy(
                src_ref=piece(chunk_src, off, i),
                dst_ref=piece(chunk_dst, off, i),
                send_sem=ssem, recv_sem=rsem,
                device_id=target, device_id_type=pl.DeviceIdType.MESH,
            )

        dn_send = [[desc(my_z + k, my_z + k, ph, i,
                         dn_s.at[k, i], dn_r.at[k, i], dn_nbr)
                    for i in range(P)] for k in range(N_Z - 1)]
        up_send = [[desc(my_z - k, my_z - k, ph, i,
                         up_s.at[k, i], up_r.at[k, i], up_nbr)
                    for i in range(P)] for k in range(N_Z - 1)]
        xl_send = [[desc((my_z + 1 + j) % N_Z, (my_z + 1 + j) % N_Z, ph, i,
                         xl_s.at[j, i], xl_r.at[j, i], partner)
                    for i in range(P)] for j in range(N_Z - 1)]
        dn_recv = [[desc(my_z + 1 + k, my_z + 1 + k, ph, i,
                         dn_s.at[k, i], dn_r.at[k, i], up_nbr)
                    for i in range(P)] for k in range(N_Z - 1)]
        up_recv = [[desc(my_z - 1 - k, my_z - 1 - k, ph, i,
                         up_s.at[k, i], up_r.at[k, i], dn_nbr)
                    for i in range(P)] for k in range(N_Z - 1)]
        xl_recv = [[desc((my_z + 1 + j) % N_Z, (my_z + 1 + j) % N_Z, qh, i,
                         xl_s.at[j, i], xl_r.at[j, i], partner)
                    for i in range(P)] for j in range(N_Z - 1)]

        def hbm_cp(chunk, off, slot):
            r = clip(chunk) * m_per + off
            return pltpu.make_async_copy(
                stg.at[pl.ds(r, half), :], out_hbm.at[pl.ds(r, half), :],
                hbm_sem.at[slot])

        own_ph_cp = hbm_cp(my_z, ph, 0)
        own_qh_cp = hbm_cp(my_z, qh, 1)
        dn_cp = [hbm_cp(my_z + 1 + k, ph, 2 + k) for k in range(N_Z - 1)]
        up_cp = [hbm_cp(my_z - 1 - k, ph, 5 + k) for k in range(N_Z - 1)]
        xl_cp = [hbm_cp((my_z + 1 + j) % N_Z, qh, 8 + j)
                 for j in range(N_Z - 1)]

        for i in range(P):
            @pl.when(has_dn)
            def _(i=i):
                dn_send[0][i].start()

            @pl.when(has_up)
            def _(i=i):
                up_send[0][i].start()

        own_ph_cp.start()

        in_cp[1].wait()
        stg[pl.ds(my_z * m_per + qh, half), :] = (
            x_vmem[pl.ds(qh, half), :].astype(jnp.bfloat16))
        own_qh_cp.start()

        for k in range(N_Z - 1):
            dn_valid = my_z + 1 + k <= N_Z - 1
            up_valid = my_z - 1 - k >= 0
            for i in range(P):
                @pl.when(dn_valid)
                def _(k=k, i=i):
                    dn_recv[k][i].wait_recv()
                    xl_send[k][i].start()
                    if k + 1 < N_Z - 1:
                        @pl.when(has_dn)
                        def _():
                            dn_send[k + 1][i].start()
                    if i == P - 1:
                        dn_cp[k].start()

                @pl.when(up_valid)
                def _(k=k, i=i):
                    up_recv[k][i].wait_recv()
                    xl_send[2 - k][i].start()
                    if k + 1 < N_Z - 1:
                        @pl.when(has_up)
                        def _():
                            up_send[k + 1][i].start()
                    if i == P - 1:
                        up_cp[k].start()

        for j in range(N_Z - 1):
            for i in range(P):
                xl_recv[j][i].wait_recv()
            xl_cp[j].start()

        for i in range(P):
            @pl.when(has_dn)
            def _(i=i):
                dn_send[0][i].wait_send()

            @pl.when(has_up)
            def _(i=i):
                up_send[0][i].wait_send()

        for k in range(N_Z - 2):
            fwd_dn = (my_z + 1 + k <= N_Z - 1) & has_dn
            fwd_up = (my_z - 1 - k >= 0) & has_up
            for i in range(P):
                @pl.when(fwd_dn)
                def _(k=k, i=i):
                    dn_send[k + 1][i].wait_send()

                @pl.when(fwd_up)
                def _(k=k, i=i):
                    up_send[k + 1][i].wait_send()

        for j in range(N_Z - 1):
            for i in range(P):
                xl_send[j][i].wait_send()

        own_ph_cp.wait()
        own_qh_cp.wait()
        for k in range(N_Z - 1):
            @pl.when(my_z + 1 + k <= N_Z - 1)
            def _(k=k):
                dn_cp[k].wait()

            @pl.when(my_z - 1 - k >= 0)
            def _(k=k):
                up_cp[k].wait()
        for j in range(N_Z - 1):
            xl_cp[j].wait()

    return pl.pallas_call(
        body,
        out_shape=jax.ShapeDtypeStruct((N_Z * m_per, n), jnp.bfloat16),
        in_specs=[pl.BlockSpec(memory_space=pl.ANY)],
        out_specs=pl.BlockSpec(memory_space=pl.ANY),
        scratch_shapes=[
            pltpu.VMEM((N_Z * m_per, n), jnp.bfloat16),
            pltpu.VMEM((m_per, n), jnp.float32),
            pltpu.SemaphoreType.DMA((N_Z - 1, P)),
            pltpu.SemaphoreType.DMA((N_Z - 1, P)),
            pltpu.SemaphoreType.DMA((N_Z - 1, P)),
            pltpu.SemaphoreType.DMA((N_Z - 1, P)),
            pltpu.SemaphoreType.DMA((N_Z - 1, P)),
            pltpu.SemaphoreType.DMA((N_Z - 1, P)),
            pltpu.SemaphoreType.DMA((2,)),
            pltpu.SemaphoreType.DMA((11,)),
        ],
        compiler_params=pltpu.CompilerParams(collective_id=0),
    )(x)


# device time: 48799 ns/iter; 1.0105x vs baseline; 1.0105x over previous
import jax
import jax.numpy as jnp
from jax import lax
from jax.experimental import pallas as pl
from jax.experimental.pallas import tpu as pltpu

N_Z = 4
P = 8


def kernel(x):
    m_per, n = x.shape
    half = m_per // 2
    rows = half // P

    def body(x_ref, out_ref,
             dn_s, up_s, xl_s, dn_r, up_r, xl_r):
        my_x = lax.axis_index("x")
        my_y = lax.axis_index("y")
        my_z = lax.axis_index("z")
        partner = (1 - my_x, my_y, my_z)
        dn_nbr = (my_x, my_y, my_z - 1)
        up_nbr = (my_x, my_y, my_z + 1)
        ph = my_x * half
        qh = (1 - my_x) * half
        has_dn = my_z > 0
        has_up = my_z < N_Z - 1

        def clip(c):
            return jnp.clip(c, 0, N_Z - 1)

        barrier = pltpu.get_barrier_semaphore()
        pl.semaphore_signal(barrier, inc=1, device_id=partner,
                            device_id_type=pl.DeviceIdType.MESH)

        @pl.when(has_dn)
        def _():
            pl.semaphore_signal(barrier, inc=1, device_id=dn_nbr,
                                device_id_type=pl.DeviceIdType.MESH)

        @pl.when(has_up)
        def _():
            pl.semaphore_signal(barrier, inc=1, device_id=up_nbr,
                                device_id_type=pl.DeviceIdType.MESH)

        out_ref[pl.ds(my_z * m_per + ph, half), :] = (
            x_ref[pl.ds(ph, half), :].astype(jnp.bfloat16))

        interior = has_dn & has_up

        @pl.when(interior)
        def _():
            pl.semaphore_wait(barrier, 3)

        @pl.when(jnp.logical_not(interior))
        def _():
            pl.semaphore_wait(barrier, 2)

        def piece(chunk, off, i):
            r = clip(chunk) * m_per + off + i * rows
            return out_ref.at[pl.ds(r, rows), :]

        def desc(chunk_src, chunk_dst, off, i, ssem, rsem, target):
            return pltpu.make_async_remote_copy(
                src_ref=piece(chunk_src, off, i),
                dst_ref=piece(chunk_dst, off, i),
                send_sem=ssem, recv_sem=rsem,
                device_id=target, device_id_type=pl.DeviceIdType.MESH,
            )

        dn_send = [[desc(my_z + k, my_z + k, ph, i,
                         dn_s.at[k, i], dn_r.at[k, i], dn_nbr)
                    for i in range(P)] for k in range(N_Z - 1)]
        up_send = [[desc(my_z - k, my_z - k, ph, i,
                         up_s.at[k, i], up_r.at[k, i], up_nbr)
                    for i in range(P)] for k in range(N_Z - 1)]
        xl_send = [[desc((my_z + 1 + j) % N_Z, (my_z + 1 + j) % N_Z, ph, i,
                         xl_s.at[j, i], xl_r.at[j, i], partner)
                    for i in range(P)] for j in range(N_Z - 1)]
        dn_recv = [[desc(my_z + 1 + k, my_z + 1 + k, ph, i,
                         dn_s.at[k, i], dn_r.at[k, i], up_nbr)
                    for i in range(P)] for k in range(N_Z - 1)]
        up_recv = [[desc(my_z - 1 - k, my_z - 1 - k, ph, i,
                         up_s.at[k, i], up_r.at[k, i], dn_nbr)
                    for i in range(P)] for k in range(N_Z - 1)]
        xl_recv = [[desc((my_z + 1 + j) % N_Z, (my_z + 1 + j) % N_Z, qh, i,
                         xl_s.at[j, i], xl_r.at[j, i], partner)
                    for i in range(P)] for j in range(N_Z - 1)]

        for i in range(P):
            @pl.when(has_dn)
            def _(i=i):
                dn_send[0][i].start()

            @pl.when(has_up)
            def _(i=i):
                up_send[0][i].start()

        out_ref[pl.ds(my_z * m_per + qh, half), :] = (
            x_ref[pl.ds(qh, half), :].astype(jnp.bfloat16))

        for k in range(N_Z - 1):
            dn_valid = my_z + 1 + k <= N_Z - 1
            up_valid = my_z - 1 - k >= 0
            for i in range(P):
                @pl.when(dn_valid)
                def _(k=k, i=i):
                    dn_recv[k][i].wait_recv()
                    xl_send[k][i].start()
                    if k + 1 < N_Z - 1:
                        @pl.when(has_dn)
                        def _():
                            dn_send[k + 1][i].start()

                @pl.when(up_valid)
                def _(k=k, i=i):
                    up_recv[k][i].wait_recv()
                    xl_send[2 - k][i].start()
                    if k + 1 < N_Z - 1:
                        @pl.when(has_up)
                        def _():
                            up_send[k + 1][i].start()

        for j in range(N_Z - 1):
            for i in range(P):
                xl_recv[j][i].wait_recv()

        for i in range(P):
            @pl.when(has_dn)
            def _(i=i):
                dn_send[0][i].wait_send()

            @pl.when(has_up)
            def _(i=i):
                up_send[0][i].wait_send()

        for k in range(N_Z - 2):
            fwd_dn = (my_z + 1 + k <= N_Z - 1) & has_dn
            fwd_up = (my_z - 1 - k >= 0) & has_up
            for i in range(P):
                @pl.when(fwd_dn)
                def _(k=k, i=i):
                    dn_send[k + 1][i].wait_send()

                @pl.when(fwd_up)
                def _(k=k, i=i):
                    up_send[k + 1][i].wait_send()

        for j in range(N_Z - 1):
            for i in range(P):
                xl_send[j][i].wait_send()

    return pl.pallas_call(
        body,
        out_shape=jax.ShapeDtypeStruct((N_Z * m_per, n), jnp.bfloat16),
        in_specs=[pl.BlockSpec(memory_space=pltpu.VMEM)],
        out_specs=pl.BlockSpec(memory_space=pltpu.VMEM),
        scratch_shapes=[
            pltpu.SemaphoreType.DMA((N_Z - 1, P)),
            pltpu.SemaphoreType.DMA((N_Z - 1, P)),
            pltpu.SemaphoreType.DMA((N_Z - 1, P)),
            pltpu.SemaphoreType.DMA((N_Z - 1, P)),
            pltpu.SemaphoreType.DMA((N_Z - 1, P)),
            pltpu.SemaphoreType.DMA((N_Z - 1, P)),
        ],
        compiler_params=pltpu.CompilerParams(collective_id=0),
    )(x)
